# Initial kernel scaffold; baseline (speedup 1.0000x reference)
import jax
import jax.numpy as jnp
from jax.experimental import pallas as pl


def kernel(x, h_tm1, c_tm1, hebb, kernel, rec_kernel, bias, w, alpha, h2mod, fanout):
    raise NotImplementedError("write your pallas kernel here")


# speedup vs baseline: 1.1345x; 1.1345x over previous
import functools

import jax
import jax.numpy as jnp
from jax.experimental import pallas as pl
from jax.experimental.pallas import tpu as pltpu

B, U, D = 256, 512, 512
BB = 8


def _gates_kernel(x_ref, h_ref, k_ref, rk_ref, b_ref, w_ref,
                  i_ref, f_ref, o_ref, cb_ref):
    x = x_ref[...]
    h = h_ref[...]
    gx = jnp.dot(x, k_ref[...], preferred_element_type=jnp.float32) + b_ref[...]
    rec = jnp.dot(h, rk_ref[...], preferred_element_type=jnp.float32)
    hw = jnp.dot(h, w_ref[...], preferred_element_type=jnp.float32)
    hs = lambda z: jnp.clip(0.2 * z + 0.5, 0.0, 1.0)
    i_ref[...] = hs(gx[:, :U] + rec[:, :U])
    f_ref[...] = hs(gx[:, U:2 * U] + rec[:, U:2 * U])
    o_ref[...] = hs(gx[:, 3 * U:] + rec[:, 3 * U:])
    cb_ref[...] = gx[:, 2 * U:3 * U] + hw


def _hebb_kernel(hebb_ref, hcol_ref, c_ref, i_ref, f_ref, o_ref, cb_ref,
                 alpha_ref, h2m_ref, fan_ref,
                 h_out_ref, c_out_ref, hebb_out_ref):
    heb = hebb_ref[...]
    hcol = hcol_ref[...]
    s = jnp.sum(heb * hcol, axis=1)
    itc = jnp.tanh(cb_ref[...] + alpha_ref[...] * s)
    c = f_ref[...] * c_ref[...] + i_ref[...] * itc
    h = o_ref[...] * jnp.tanh(c)
    eta = jnp.tanh(jnp.sum(h * h2m_ref[...], axis=1, keepdims=True))
    g = (eta * fan_ref[...]) * itc
    h_out_ref[...] = h
    c_out_ref[...] = c
    hebb_out_ref[...] = jnp.clip(heb + hcol * g[:, None, :], -2.0, 2.0)


@jax.jit
def kernel(x, h_tm1, c_tm1, hebb, kernel, rec_kernel, bias, w, alpha, h2mod,
           fanout):
    f32 = jnp.float32
    i_g, f_g, o_g, cb = pl.pallas_call(
        _gates_kernel,
        out_shape=[jax.ShapeDtypeStruct((B, U), f32)] * 4,
        name="lstm_gates",
    )(x, h_tm1, kernel, rec_kernel, bias.reshape(1, 4 * U), w)

    hcol = h_tm1.reshape(B, U, 1)
    vec_spec = pl.BlockSpec((BB, U), lambda b: (b, 0))
    row_spec = pl.BlockSpec((1, U), lambda b: (0, 0))
    cube_spec = pl.BlockSpec((BB, U, U), lambda b: (b, 0, 0))
    h_out, c_out, hebb_out = pl.pallas_call(
        _hebb_kernel,
        grid=(B // BB,),
        in_specs=[
            cube_spec,
            pl.BlockSpec((BB, U, 1), lambda b: (b, 0, 0)),
            vec_spec, vec_spec, vec_spec, vec_spec, vec_spec,
            row_spec, row_spec, row_spec,
        ],
        out_specs=[vec_spec, vec_spec, cube_spec],
        out_shape=[
            jax.ShapeDtypeStruct((B, U), f32),
            jax.ShapeDtypeStruct((B, U), f32),
            jax.ShapeDtypeStruct((B, U, U), f32),
        ],
        compiler_params=pltpu.CompilerParams(
            dimension_semantics=("parallel",),
            vmem_limit_bytes=52 * 1024 * 1024,
        ),
        name="hebb_update",
    )(hebb, hcol, c_tm1, i_g, f_g, o_g, cb,
      alpha.reshape(1, U), h2mod.reshape(1, U), fanout)
    return h_out, c_out, hebb_out


# speedup vs baseline: 1.1489x; 1.0127x over previous
import jax
import jax.numpy as jnp
from jax.experimental import pallas as pl
from jax.experimental.pallas import tpu as pltpu

B, U, D = 256, 512, 512
BB = 8


def _cell_kernel(x_ref, hcol_ref, h_ref, c_ref, hebb_ref, k_ref, rk_ref,
                 b_ref, w_ref, alpha_ref, h2m_ref, fan_ref,
                 h_out_ref, c_out_ref, hebb_out_ref):
    x = x_ref[...]
    h = h_ref[...]
    gx = jnp.dot(x, k_ref[...], preferred_element_type=jnp.float32) + b_ref[...]
    rec = jnp.dot(h, rk_ref[...], preferred_element_type=jnp.float32)
    hw = jnp.dot(h, w_ref[...], preferred_element_type=jnp.float32)
    hs = lambda z: jnp.clip(0.2 * z + 0.5, 0.0, 1.0)
    i_g = hs(gx[:, :U] + rec[:, :U])
    f_g = hs(gx[:, U:2 * U] + rec[:, U:2 * U])
    o_g = hs(gx[:, 3 * U:] + rec[:, 3 * U:])
    cb = gx[:, 2 * U:3 * U] + hw

    heb = hebb_ref[...]
    hcol = hcol_ref[...]
    s = jnp.sum(heb * hcol, axis=1)
    itc = jnp.tanh(cb + alpha_ref[...] * s)
    c = f_g * c_ref[...] + i_g * itc
    h_new = o_g * jnp.tanh(c)
    eta = jnp.tanh(jnp.sum(h_new * h2m_ref[...], axis=1, keepdims=True))
    g = (eta * fan_ref[...]) * itc
    h_out_ref[...] = h_new
    c_out_ref[...] = c
    hebb_out_ref[...] = jnp.clip(heb + hcol * g[:, None, :], -2.0, 2.0)


@jax.jit
def kernel(x, h_tm1, c_tm1, hebb, kernel, rec_kernel, bias, w, alpha, h2mod,
           fanout):
    f32 = jnp.float32
    hcol = h_tm1.reshape(B, U, 1)
    vec_spec = pl.BlockSpec((BB, U), lambda b: (b, 0))
    row_spec = pl.BlockSpec((1, U), lambda b: (0, 0))
    cube_spec = pl.BlockSpec((BB, U, U), lambda b: (b, 0, 0))
    full = lambda shape: pl.BlockSpec(shape, lambda b: (0,) * len(shape))
    h_out, c_out, hebb_out = pl.pallas_call(
        _cell_kernel,
        grid=(B // BB,),
        in_specs=[
            pl.BlockSpec((BB, D), lambda b: (b, 0)),
            pl.BlockSpec((BB, U, 1), lambda b: (b, 0, 0)),
            vec_spec,
            vec_spec,
            cube_spec,
            full((D, 4 * U)),
            full((U, 4 * U)),
            pl.BlockSpec((1, 4 * U), lambda b: (0, 0)),
            full((U, U)),
            row_spec, row_spec, row_spec,
        ],
        out_specs=[vec_spec, vec_spec, cube_spec],
        out_shape=[
            jax.ShapeDtypeStruct((B, U), f32),
            jax.ShapeDtypeStruct((B, U), f32),
            jax.ShapeDtypeStruct((B, U, U), f32),
        ],
        compiler_params=pltpu.CompilerParams(
            dimension_semantics=("parallel",),
            vmem_limit_bytes=56 * 1024 * 1024,
        ),
        name="lstm_cell_nm",
    )(x, hcol, h_tm1, c_tm1, hebb, kernel, rec_kernel,
      bias.reshape(1, 4 * U), w, alpha.reshape(1, U), h2mod.reshape(1, U),
      fanout)
    return h_out, c_out, hebb_out


# speedup vs baseline: 1.1553x; 1.0055x over previous
import jax
import jax.numpy as jnp
from jax.experimental import pallas as pl
from jax.experimental.pallas import tpu as pltpu

B, U, D = 256, 512, 512
BB = 8


def _cell_kernel(x_ref, hcol_ref, h_ref, c_ref, hebb_ref, k_ref, rk_ref,
                 b_ref, w_ref, alpha_ref, h2m_ref, fan_ref,
                 h_out_ref, c_out_ref, hebb_out_ref):
    x = x_ref[...]
    h = h_ref[...]
    gx = jnp.dot(x, k_ref[...], preferred_element_type=jnp.float32) + b_ref[...]
    rec = jnp.dot(h, rk_ref[...], preferred_element_type=jnp.float32)
    hw = jnp.dot(h, w_ref[...], preferred_element_type=jnp.float32)
    hs = lambda z: jnp.clip(0.2 * z + 0.5, 0.0, 1.0)
    i_g = hs(gx[:, :U] + rec[:, :U])
    f_g = hs(gx[:, U:2 * U] + rec[:, U:2 * U])
    o_g = hs(gx[:, 3 * U:] + rec[:, 3 * U:])
    cb = gx[:, 2 * U:3 * U] + hw

    CH = 32
    s = jnp.zeros((BB, U), jnp.float32)
    for k in range(0, U, CH):
        s = s + jnp.sum(hebb_ref[:, k:k + CH, :] * hcol_ref[:, k:k + CH, :],
                        axis=1)
    itc = jnp.tanh(cb + alpha_ref[...] * s)
    c = f_g * c_ref[...] + i_g * itc
    h_new = o_g * jnp.tanh(c)
    eta = jnp.tanh(jnp.sum(h_new * h2m_ref[...], axis=1, keepdims=True))
    g = (eta * fan_ref[...]) * itc
    grow = g[:, None, :]
    h_out_ref[...] = h_new
    c_out_ref[...] = c
    for k in range(0, U, CH):
        hebb_out_ref[:, k:k + CH, :] = jnp.clip(
            hebb_ref[:, k:k + CH, :] + hcol_ref[:, k:k + CH, :] * grow,
            -2.0, 2.0)


@jax.jit
def kernel(x, h_tm1, c_tm1, hebb, kernel, rec_kernel, bias, w, alpha, h2mod,
           fanout):
    f32 = jnp.float32
    hcol = h_tm1.reshape(B, U, 1)
    vec_spec = pl.BlockSpec((BB, U), lambda b: (b, 0))
    row_spec = pl.BlockSpec((1, U), lambda b: (0, 0))
    cube_spec = pl.BlockSpec((BB, U, U), lambda b: (b, 0, 0))
    full = lambda shape: pl.BlockSpec(shape, lambda b: (0,) * len(shape))
    h_out, c_out, hebb_out = pl.pallas_call(
        _cell_kernel,
        grid=(B // BB,),
        in_specs=[
            pl.BlockSpec((BB, D), lambda b: (b, 0)),
            pl.BlockSpec((BB, U, 1), lambda b: (b, 0, 0)),
            vec_spec,
            vec_spec,
            cube_spec,
            full((D, 4 * U)),
            full((U, 4 * U)),
            pl.BlockSpec((1, 4 * U), lambda b: (0, 0)),
            full((U, U)),
            row_spec, row_spec, row_spec,
        ],
        out_specs=[vec_spec, vec_spec, cube_spec],
        out_shape=[
            jax.ShapeDtypeStruct((B, U), f32),
            jax.ShapeDtypeStruct((B, U), f32),
            jax.ShapeDtypeStruct((B, U, U), f32),
        ],
        compiler_params=pltpu.CompilerParams(
            dimension_semantics=("parallel",),
            vmem_limit_bytes=56 * 1024 * 1024,
        ),
        name="lstm_cell_nm",
    )(x, hcol, h_tm1, c_tm1, hebb, kernel, rec_kernel,
      bias.reshape(1, 4 * U), w, alpha.reshape(1, U), h2mod.reshape(1, U),
      fanout)
    return h_out, c_out, hebb_out


# speedup vs baseline: 1.4901x; 1.2898x over previous
import jax
import jax.numpy as jnp
from jax.experimental import pallas as pl
from jax.experimental.pallas import tpu as pltpu

B, U, D = 256, 512, 512
BB = 8


def _cell_kernel(x_ref, h_ref, c_ref, hebb_ref, k_ref, rk_ref,
                 b_ref, w_ref, alpha_ref, h2m_ref, fan_ref,
                 h_out_ref, c_out_ref, hebb_out_ref):
    x = x_ref[...]
    h = h_ref[...]
    hcol = h.reshape(BB, U, 1)
    gx = jnp.dot(x, k_ref[...], preferred_element_type=jnp.float32) + b_ref[...]
    rec = jnp.dot(h, rk_ref[...], preferred_element_type=jnp.float32)
    hw = jnp.dot(h, w_ref[...], preferred_element_type=jnp.float32)
    hs = lambda z: jnp.clip(0.2 * z + 0.5, 0.0, 1.0)
    i_g = hs(gx[:, :U] + rec[:, :U])
    f_g = hs(gx[:, U:2 * U] + rec[:, U:2 * U])
    o_g = hs(gx[:, 3 * U:] + rec[:, 3 * U:])
    cb = gx[:, 2 * U:3 * U] + hw

    CH = 32
    s = jnp.zeros((BB, U), jnp.float32)
    for k in range(0, U, CH):
        s = s + jnp.sum(hebb_ref[:, k:k + CH, :] * hcol[:, k:k + CH, :],
                        axis=1)
    itc = jnp.tanh(cb + alpha_ref[...] * s)
    c = f_g * c_ref[...] + i_g * itc
    h_new = o_g * jnp.tanh(c)
    eta = jnp.tanh(jnp.sum(h_new * h2m_ref[...], axis=1, keepdims=True))
    g = (eta * fan_ref[...]) * itc
    grow = g[:, None, :]
    h_out_ref[...] = h_new
    c_out_ref[...] = c
    for k in range(0, U, CH):
        hebb_out_ref[:, k:k + CH, :] = jnp.clip(
            hebb_ref[:, k:k + CH, :] + hcol[:, k:k + CH, :] * grow,
            -2.0, 2.0)


@jax.jit
def kernel(x, h_tm1, c_tm1, hebb, kernel, rec_kernel, bias, w, alpha, h2mod,
           fanout):
    f32 = jnp.float32
    vec_spec = pl.BlockSpec((BB, U), lambda b: (b, 0))
    row_spec = pl.BlockSpec((1, U), lambda b: (0, 0))
    cube_spec = pl.BlockSpec((BB, U, U), lambda b: (b, 0, 0))
    full = lambda shape: pl.BlockSpec(shape, lambda b: (0,) * len(shape))
    h_out, c_out, hebb_out = pl.pallas_call(
        _cell_kernel,
        grid=(B // BB,),
        in_specs=[
            pl.BlockSpec((BB, D), lambda b: (b, 0)),
            vec_spec,
            vec_spec,
            cube_spec,
            full((D, 4 * U)),
            full((U, 4 * U)),
            pl.BlockSpec((1, 4 * U), lambda b: (0, 0)),
            full((U, U)),
            row_spec, row_spec, row_spec,
        ],
        out_specs=[vec_spec, vec_spec, cube_spec],
        out_shape=[
            jax.ShapeDtypeStruct((B, U), f32),
            jax.ShapeDtypeStruct((B, U), f32),
            jax.ShapeDtypeStruct((B, U, U), f32),
        ],
        compiler_params=pltpu.CompilerParams(
            dimension_semantics=("parallel",),
            vmem_limit_bytes=56 * 1024 * 1024,
        ),
        name="lstm_cell_nm",
    )(x, h_tm1, c_tm1, hebb, kernel, rec_kernel,
      bias.reshape(1, 4 * U), w, alpha.reshape(1, U), h2mod.reshape(1, U),
      fanout)
    return h_out, c_out, hebb_out
